# baseline (speedup 1.0000x reference)
import functools

import jax
import jax.numpy as jnp
from jax import lax
from jax.experimental import pallas as pl
from jax.experimental.pallas import tpu as pltpu
from jax.experimental.pallas import tpu_sc as plsc

L = 16
NC = 2
NS = 16
NW = NC * NS
B = 16384
D = 64
BPW = B // NW
GCH = 128
NCH = BPW // GCH


def _gmf_body(uidx_hbm, iidx_hbm, utab_hbm, itab_hbm, w_hbm, b_hbm,
              out_hbm, uidx_v, iidx_v, urows_v, irows_v, w_v, b_v,
              out_v, gsem):
    wid = lax.axis_index("s") * NC + lax.axis_index("c")
    base = wid * BPW

    pltpu.sync_copy(uidx_hbm.at[pl.ds(base, BPW)], uidx_v)
    pltpu.sync_copy(iidx_hbm.at[pl.ds(base, BPW)], iidx_v)
    pltpu.sync_copy(w_hbm, w_v)
    pltpu.sync_copy(b_hbm, b_v)

    copies = []
    for c in range(NCH):
        sl = pl.ds(c * GCH, GCH)
        copies.append(pltpu.async_copy(
            utab_hbm.at[uidx_v.at[sl]], urows_v.at[sl], gsem))
        copies.append(pltpu.async_copy(
            itab_hbm.at[iidx_v.at[sl]], irows_v.at[sl], gsem))
    for cp in copies:
        cp.wait()

    w0 = w_v[pl.ds(0, L)]
    w1 = w_v[pl.ds(L, L)]
    w2 = w_v[pl.ds(2 * L, L)]
    w3 = w_v[pl.ds(3 * L, L)]
    bias = b_v[...]
    lane = lax.iota(jnp.int32, L)
    perms = [lane ^ s for s in (8, 4, 2, 1)]

    def lanesum(v):
        for p in perms:
            v = v + v.at[p].get(mode="promise_in_bounds", unique_indices=True)
        return v

    def group_body(j, carry):
        res = jnp.zeros((L,), jnp.float32)
        for k in range(L):
            i = j * L + k
            u0 = urows_v[i, pl.ds(0, L)]
            u1 = urows_v[i, pl.ds(L, L)]
            u2 = urows_v[i, pl.ds(2 * L, L)]
            u3 = urows_v[i, pl.ds(3 * L, L)]
            v0 = irows_v[i, pl.ds(0, L)]
            v1 = irows_v[i, pl.ds(L, L)]
            v2 = irows_v[i, pl.ds(2 * L, L)]
            v3 = irows_v[i, pl.ds(3 * L, L)]
            acc = ((u0 * v0) * w0 + (u1 * v1) * w1
                   + (u2 * v2) * w2 + (u3 * v3) * w3)
            res = jnp.where(lane == k, lanesum(acc), res)
        x = res + bias
        out_v[pl.ds(j * L, L)] = 1.0 / (1.0 + jnp.exp(-x))
        return carry

    lax.fori_loop(0, BPW // L, group_body, 0)

    pltpu.sync_copy(out_v, out_hbm.at[pl.ds(base, BPW)])


@functools.partial(jax.jit, static_argnames=())
def _gmf(user_indices, item_indices, user_table, item_table, w64, b16):
    mesh = plsc.VectorSubcoreMesh(core_axis_name="c", subcore_axis_name="s")
    run = functools.partial(
        pl.kernel,
        mesh=mesh,
        compiler_params=pltpu.CompilerParams(use_tc_tiling_on_sc=False),
        out_type=jax.ShapeDtypeStruct((B,), jnp.float32),
        scratch_types=[
            pltpu.VMEM((BPW,), jnp.int32),
            pltpu.VMEM((BPW,), jnp.int32),
            pltpu.VMEM((BPW, D), jnp.float32),
            pltpu.VMEM((BPW, D), jnp.float32),
            pltpu.VMEM((D,), jnp.float32),
            pltpu.VMEM((L,), jnp.float32),
            pltpu.VMEM((BPW,), jnp.float32),
            pltpu.SemaphoreType.DMA,
        ],
    )(_gmf_body)
    return run(user_indices, item_indices, user_table, item_table, w64, b16)


def kernel(user_indices, item_indices, user_table, item_table, W, b):
    w64 = jnp.reshape(W.astype(jnp.float32), (D,))
    b16 = jnp.full((L,), b[0], dtype=jnp.float32)
    out = _gmf(user_indices.astype(jnp.int32), item_indices.astype(jnp.int32),
               user_table, item_table, w64, b16)
    return jnp.reshape(out, (B, 1))
